# baseline (speedup 1.0000x reference)
import functools

import jax
import jax.numpy as jnp
from jax import lax
from jax.experimental import pallas as pl
from jax.experimental.pallas import tpu as pltpu
import jax.experimental.pallas.tpu_sc as plsc

_NGRAM_RANGE = (3, 4, 5, 6)
_MAX_NGRAM = 6
_NGRAM_VOCAB = 65536
_DIM = 32

_NC = 2
_NS = 16
_NW = _NC * _NS
_LANES = 16

_PAD = 8
_CHUNK = 64
_ROW = 128


def _sc_body(seq_len, chunks_per_worker, bytes_hbm, btab_hbm, ntab_hbm,
             out_hbm, bytes_v, btab_v, idx_v, rows_v, obuf,
             sem_g0, sem_g1, sem_o0, sem_o1):
    i32 = jnp.int32
    tokens_per_worker = chunks_per_worker * _CHUNK
    wid = (lax.axis_index("s") * _NC + lax.axis_index("c")).astype(jnp.int32)
    workers_per_seq = seq_len // tokens_per_worker
    q = wid // workers_per_seq
    pb = (wid % workers_per_seq) * tokens_per_worker
    at_seq_start = pb == 0

    pltpu.sync_copy(btab_hbm, btab_v)
    off = q * seq_len + pb

    @pl.when(at_seq_start)
    def _():
        bytes_v[pl.ds(0, 2 * _PAD)] = jnp.zeros((2 * _PAD,), jnp.int32)
        pltpu.sync_copy(bytes_hbm.at[pl.ds(off, tokens_per_worker)],
                        bytes_v.at[pl.ds(_PAD, tokens_per_worker)])

    @pl.when(jnp.logical_not(at_seq_start))
    def _():
        pltpu.sync_copy(bytes_hbm.at[pl.ds(off - _PAD,
                                           tokens_per_worker + _PAD)],
                        bytes_v.at[pl.ds(0, tokens_per_worker + _PAD)])

    sem_g = (sem_g0, sem_g1)
    sem_o = (sem_o0, sem_o1)

    def do_hash(c):
        b = i32(c % 2)
        for g in range(_CHUNK // _LANES):
            off = _PAD + c * _CHUNK + g * _LANES
            b0 = bytes_v[pl.ds(off, _LANES)]
            b1 = bytes_v[pl.ds(off - 1, _LANES)]
            b2 = bytes_v[pl.ds(off - 2, _LANES)]
            b3 = bytes_v[pl.ds(off - 3, _LANES)]
            b4 = bytes_v[pl.ds(off - 4, _LANES)]
            b5 = bytes_v[pl.ds(off - 5, _LANES)]
            h3 = (b0 * 513 + b1 * 257 + b2) & 0xFFFF
            h4 = (b3 + h3 * 257) & 0xFFFF
            h5 = (b4 + h4 * 257) & 0xFFFF
            h6 = (b5 + h5 * 257) & 0xFFFF
            gs = pl.ds(g * _LANES, _LANES)
            for r, h in enumerate((h3, h4, h5, h6)):
                idx_v[b, i32(r), gs] = (h >> 2) + r * (_NGRAM_VOCAB // 4)
                idx_v[b, i32(4 + r), gs] = (h & 3) * _DIM

    def fire_gathers(c):
        b = c % 2
        bi = i32(b)
        cpys = []
        for r in range(4):
            cpys.append(pltpu.async_copy(
                ntab_hbm.at[idx_v.at[bi, i32(r)]],
                rows_v.at[bi, i32(r)], sem_g[b]))
        return cpys

    out_cpys = {}
    do_hash(0)
    gathers = fire_gathers(0)

    for c in range(chunks_per_worker):
        b = c % 2
        bi = i32(b)

        if c + 1 < chunks_per_worker:
            do_hash(c + 1)
            next_gathers = fire_gathers(c + 1)

        for cp in gathers:
            cp.wait()
        if c + 1 < chunks_per_worker:
            gathers = next_gathers

        if c == 0:
            @pl.when(at_seq_start)
            def _():
                zeros = jnp.zeros((_LANES,), jnp.float32)
                for p in range(_MAX_NGRAM - 1):
                    for r in range(4):
                        if p + 1 < _NGRAM_RANGE[r]:
                            for h in range(_ROW // _LANES):
                                rows_v[bi, i32(r), i32(p),
                                       pl.ds(h * _LANES, _LANES)] = zeros

        if c >= 2:
            out_cpys.pop(c - 2).wait()

        tok0 = i32(_PAD + c * _CHUNK)
        iota = lax.iota(jnp.int32, _LANES)

        @pl.loop(i32(0), i32(_CHUNK))
        def _(t):
            tsp = jnp.zeros((_LANES,), jnp.int32) + t
            osp = [plsc.load_gather(idx_v.at[bi, i32(4 + r)], [tsp])
                   for r in range(4)]
            bsp = plsc.load_gather(bytes_v, [tsp + tok0])
            for h in range(_DIM // _LANES):
                p = bsp * _DIM + iota + i32(h * _LANES)
                base = plsc.load_gather(btab_v, [p >> 7, p & 127])
                col = [o + iota + i32(h * _LANES) for o in osp]
                s01 = (plsc.load_gather(rows_v.at[bi, i32(0)], [tsp, col[0]])
                       + plsc.load_gather(rows_v.at[bi, i32(1)],
                                          [tsp, col[1]]))
                s23 = (plsc.load_gather(rows_v.at[bi, i32(2)], [tsp, col[2]])
                       + plsc.load_gather(rows_v.at[bi, i32(3)],
                                          [tsp, col[3]]))
                obuf[bi, t, pl.ds(h * _LANES, _LANES)] = (
                    base + (s01 + s23) * 0.25)

        out0 = wid * tokens_per_worker + c * _CHUNK
        out_cpys[c] = pltpu.async_copy(
            obuf.at[bi], out_hbm.at[pl.ds(out0, _CHUNK), :], sem_o[b])

    for cp in out_cpys.values():
        cp.wait()


def kernel(byte_ids, byte_table, ngram_table):
    B, S = byte_ids.shape
    dim = byte_table.shape[-1]
    n_tokens = B * S
    tokens_per_worker = n_tokens // _NW
    chunks_per_worker = tokens_per_worker // _CHUNK
    pack = _ROW // dim

    b32 = jnp.bitwise_and(byte_ids, 1023).astype(jnp.int32).reshape(-1)
    btab = byte_table.astype(jnp.float32).reshape(-1, _ROW)
    ntab = ngram_table.astype(jnp.float32).reshape(-1, _ROW)

    mesh = plsc.VectorSubcoreMesh(
        core_axis_name="c", subcore_axis_name="s",
        num_cores=_NC, num_subcores=_NS)

    body = functools.partial(_sc_body, S, chunks_per_worker)
    out = pl.kernel(
        body,
        out_type=jax.ShapeDtypeStruct((n_tokens, dim), jnp.float32),
        mesh=mesh,
        scratch_types=[
            pltpu.VMEM((tokens_per_worker + 2 * _PAD,), jnp.int32),
            pltpu.VMEM((byte_table.shape[0] // pack, _ROW),
                       jnp.float32),
            pltpu.VMEM((2, 8, _CHUNK), jnp.int32),
            pltpu.VMEM((2, 4, _CHUNK, _ROW), jnp.float32),
            pltpu.VMEM((2, _CHUNK, dim), jnp.float32),
            pltpu.SemaphoreType.DMA,
            pltpu.SemaphoreType.DMA,
            pltpu.SemaphoreType.DMA,
            pltpu.SemaphoreType.DMA,
        ],
        compiler_params=pltpu.CompilerParams(
            use_tc_tiling_on_sc=True, needs_layout_passes=False),
    )(b32, btab, ntab)
    return out.reshape(B, S, dim)
